# baseline (speedup 1.0000x reference)
import jax
import jax.numpy as jnp
from jax.experimental import pallas as pl
from jax.experimental.pallas import tpu as pltpu

_BT = 512
_KB = 1024


def _router_block(x_ref, w_ref, b_ref, o_ref, acc):
    kk = pl.program_id(1)
    nk = pl.num_programs(1)
    part = jnp.dot(x_ref[...], w_ref[...], preferred_element_type=jnp.float32)

    @pl.when(kk == 0)
    def _():
        acc[...] = part

    @pl.when(kk != 0)
    def _():
        acc[...] += part

    @pl.when(kk == nk - 1)
    def _():
        logits = acc[...] + b_ref[...].reshape(1, -1)
        m = jnp.max(logits, axis=-1, keepdims=True)
        e = jnp.exp(logits - m)
        o_ref[...] = e * (1.0 / jnp.sum(e, axis=-1, keepdims=True))


def kernel(x, W, b):
    n, k = x.shape
    ne = W.shape[1]
    return pl.pallas_call(
        _router_block,
        grid=(n // _BT, k // _KB),
        in_specs=[
            pl.BlockSpec((_BT, _KB), lambda i, kk: (i, kk)),
            pl.BlockSpec((_KB, ne), lambda i, kk: (kk, 0)),
            pl.BlockSpec((ne,), lambda i, kk: (0,)),
        ],
        out_specs=pl.BlockSpec((_BT, ne), lambda i, kk: (i, 0)),
        out_shape=jax.ShapeDtypeStruct((n, ne), jnp.float32),
        scratch_shapes=[pltpu.VMEM((_BT, ne), jnp.float32)],
        compiler_params=pltpu.CompilerParams(
            dimension_semantics=("parallel", "arbitrary"),
        ),
    )(x, W, b)


# speedup vs baseline: 1.7145x; 1.7145x over previous
import jax
import jax.numpy as jnp
from jax.experimental import pallas as pl
from jax.experimental.pallas import tpu as pltpu

_CH = 256
_NBUF = 4
_G = 8


def _router_body(x_hbm, w_ref, b_ref, o_hbm, xbuf, obuf, insem, outsem):
    n = x_hbm.shape[0]
    nchunks = n // _CH
    ngroups = nchunks // _G

    def in_copy(i, slot):
        return pltpu.make_async_copy(
            x_hbm.at[pl.ds(i * _CH, _CH), :], xbuf.at[slot], insem.at[slot]
        )

    def out_copy(g, gslot):
        return pltpu.make_async_copy(
            obuf.at[gslot],
            o_hbm.at[pl.ds(g * (_G * _CH), _G * _CH), :],
            outsem.at[gslot],
        )

    for j in range(_NBUF):
        in_copy(j, j).start()

    def group(g, carry):
        gslot = jax.lax.rem(g, 2)

        @pl.when(g >= 2)
        def _():
            out_copy(g - 2, gslot).wait()

        for s in range(_G):
            i = g * _G + s
            in_copy(i, s % _NBUF).wait()
            logits = jnp.dot(
                xbuf[s % _NBUF], w_ref[...], preferred_element_type=jnp.float32
            )
            logits = logits + b_ref[...].reshape(1, -1)
            m = jnp.max(logits, axis=-1, keepdims=True)
            e = jnp.exp(logits - m)
            obuf[gslot, pl.ds(s * _CH, _CH), :] = e * (
                1.0 / jnp.sum(e, axis=-1, keepdims=True)
            )

            @pl.when(i + _NBUF < nchunks)
            def _():
                in_copy(i + _NBUF, s % _NBUF).start()

        out_copy(g, gslot).start()
        return carry

    jax.lax.fori_loop(0, ngroups, group, 0, unroll=False)

    out_copy(ngroups - 2, jax.lax.rem(ngroups - 2, 2)).wait()
    out_copy(ngroups - 1, jax.lax.rem(ngroups - 1, 2)).wait()


def kernel(x, W, b):
    n, k = x.shape
    ne = W.shape[1]
    return pl.pallas_call(
        _router_body,
        in_specs=[
            pl.BlockSpec(memory_space=pltpu.MemorySpace.HBM),
            pl.BlockSpec(memory_space=pltpu.MemorySpace.VMEM),
            pl.BlockSpec(memory_space=pltpu.MemorySpace.VMEM),
        ],
        out_specs=pl.BlockSpec(memory_space=pltpu.MemorySpace.HBM),
        out_shape=jax.ShapeDtypeStruct((n, ne), jnp.float32),
        scratch_shapes=[
            pltpu.VMEM((_NBUF, _CH, k), jnp.float32),
            pltpu.VMEM((2, _G * _CH, ne), jnp.float32),
            pltpu.SemaphoreType.DMA((_NBUF,)),
            pltpu.SemaphoreType.DMA((2,)),
        ],
    )(x, W, b)


# speedup vs baseline: 1.9038x; 1.1104x over previous
import jax
import jax.numpy as jnp
from jax.experimental import pallas as pl
from jax.experimental.pallas import tpu as pltpu

_CH = 256
_NBUF = 4
_NQ = 4


def _router_body(x_hbm, w_ref, b_ref, o_hbm, xbuf, obuf, insem, outsem):
    n = x_hbm.shape[0]
    nchunks = n // _CH
    qchunks = nchunks // _NQ
    qrows = n // _NQ

    def in_copy(i, slot):
        return pltpu.make_async_copy(
            x_hbm.at[pl.ds(i * _CH, _CH), :], xbuf.at[slot], insem.at[slot]
        )

    def out_copy(q):
        return pltpu.make_async_copy(
            obuf.at[pl.ds(q * qrows, qrows), :],
            o_hbm.at[pl.ds(q * qrows, qrows), :],
            outsem.at[q],
        )

    for j in range(_NBUF):
        in_copy(j, j).start()

    def step(i, carry):
        slot = jax.lax.rem(i, _NBUF)
        in_copy(i, slot).wait()
        logits = jnp.dot(
            xbuf[slot], w_ref[...], preferred_element_type=jnp.float32
        )
        logits = logits + b_ref[...].reshape(1, -1)
        m = jnp.max(logits, axis=-1, keepdims=True)
        e = jnp.exp(logits - m)
        obuf[pl.ds(i * _CH, _CH), :] = e * (
            1.0 / jnp.sum(e, axis=-1, keepdims=True)
        )

        @pl.when(i + _NBUF < nchunks)
        def _():
            in_copy(i + _NBUF, slot).start()

        for q in range(_NQ):

            @pl.when(i == (q + 1) * qchunks - 1)
            def _():
                out_copy(q).start()

        return carry

    jax.lax.fori_loop(0, nchunks, step, 0, unroll=False)

    for q in range(_NQ):
        out_copy(q).wait()


def kernel(x, W, b):
    n, k = x.shape
    ne = W.shape[1]
    return pl.pallas_call(
        _router_body,
        in_specs=[
            pl.BlockSpec(memory_space=pltpu.MemorySpace.HBM),
            pl.BlockSpec(memory_space=pltpu.MemorySpace.VMEM),
            pl.BlockSpec(memory_space=pltpu.MemorySpace.VMEM),
        ],
        out_specs=pl.BlockSpec(memory_space=pltpu.MemorySpace.HBM),
        out_shape=jax.ShapeDtypeStruct((n, ne), jnp.float32),
        scratch_shapes=[
            pltpu.VMEM((_NBUF, _CH, k), jnp.float32),
            pltpu.VMEM((n, ne), jnp.float32),
            pltpu.SemaphoreType.DMA((_NBUF,)),
            pltpu.SemaphoreType.DMA((_NQ,)),
        ],
    )(x, W, b)


# speedup vs baseline: 1.9048x; 1.0005x over previous
import jax
import jax.numpy as jnp
from jax.experimental import pallas as pl
from jax.experimental.pallas import tpu as pltpu

_CH = 256
_NBUF = 4


def _router_body(x_hbm, w_ref, b_ref, o_ref, xbuf, insem):
    n = x_hbm.shape[0]
    nchunks = n // _CH

    def in_copy(i, slot):
        return pltpu.make_async_copy(
            x_hbm.at[pl.ds(i * _CH, _CH), :], xbuf.at[slot], insem.at[slot]
        )

    for j in range(_NBUF - 1):
        in_copy(j, j).start()

    def step(i, carry):
        slot = jax.lax.rem(i, _NBUF)
        ahead = i + _NBUF - 1

        @pl.when(ahead < nchunks)
        def _():
            in_copy(ahead, jax.lax.rem(ahead, _NBUF)).start()

        in_copy(i, slot).wait()
        logits = jnp.dot(
            xbuf[slot], w_ref[...], preferred_element_type=jnp.float32
        )
        logits = logits + b_ref[...].reshape(1, -1)
        m = jnp.max(logits, axis=-1, keepdims=True)
        e = jnp.exp(logits - m)
        o_ref[pl.ds(i * _CH, _CH), :] = e * (
            1.0 / jnp.sum(e, axis=-1, keepdims=True)
        )
        return carry

    jax.lax.fori_loop(0, nchunks, step, 0, unroll=False)


def kernel(x, W, b):
    n, k = x.shape
    ne = W.shape[1]
    return pl.pallas_call(
        _router_body,
        in_specs=[
            pl.BlockSpec(memory_space=pltpu.MemorySpace.HBM),
            pl.BlockSpec(memory_space=pltpu.MemorySpace.VMEM),
            pl.BlockSpec(memory_space=pltpu.MemorySpace.VMEM),
        ],
        out_specs=pl.BlockSpec(memory_space=pltpu.MemorySpace.VMEM),
        out_shape=jax.ShapeDtypeStruct((n, ne), jnp.float32),
        scratch_shapes=[
            pltpu.VMEM((_NBUF, _CH, k), jnp.float32),
            pltpu.SemaphoreType.DMA((_NBUF,)),
        ],
    )(x, W, b)
